# Initial kernel scaffold; baseline (speedup 1.0000x reference)
import jax
import jax.numpy as jnp
from jax.experimental import pallas as pl


def kernel(x, edge_index, edge_attr, node_tabs, edge_tabs, W1, b1, W2, b2, gamma, beta):
    raise NotImplementedError("write your pallas kernel here")


# speedup vs baseline: 5.5556x; 5.5556x over previous
import functools

import jax
import jax.numpy as jnp
from jax import lax
from jax.experimental import pallas as pl
from jax.experimental.pallas import tpu as pltpu
from jax.experimental.pallas import tpu_sc as plsc

N = 10000
D = 128
E = 320000
L = 5

NC = 2
NS = 16
NW = NC * NS

NPW = 320
N_PAD = NW * NPW

N_ACC = 10112
RPT = N_ACC // NS

K = 128

CH_E = 79
E_PAD = NW * CH_E * K

_MESH = plsc.VectorSubcoreMesh(core_axis_name="c", subcore_axis_name="s")


@functools.partial(
    pl.kernel,
    out_type=jax.ShapeDtypeStruct((N_PAD, D), jnp.float32),
    mesh=_MESH,
    scratch_types=[
        pltpu.VMEM((1, 128), jnp.int32),
        pltpu.VMEM((128, D), jnp.float32),
        pltpu.VMEM((16, D), jnp.float32),
        pltpu.SemaphoreType.DMA,
    ],
)
def _embed(tab_hbm, idx_hbm, out_hbm, idxv, rows, outv, sem):
    wid = lax.axis_index("s") * NC + lax.axis_index("c")

    def chunk(i, carry):
        node_off = wid * NPW + i * 16
        pltpu.sync_copy(idx_hbm.at[pl.ds(node_off * 8, 128)], idxv.at[0])
        pltpu.async_copy(tab_hbm.at[idxv.at[0]], rows, sem).wait()
        for j in range(16):
            for q in range(8):
                acc = rows[8 * j, pl.ds(16 * q, 16)]
                for t in range(1, 8):
                    acc = acc + rows[8 * j + t, pl.ds(16 * q, 16)]
                outv[j, pl.ds(16 * q, 16)] = acc
        pltpu.sync_copy(outv, out_hbm.at[pl.ds(node_off, 16)])
        return carry

    lax.fori_loop(0, NPW // 16, chunk, 0)


def _make_segsum(W, n_chunks):
    per_tile = n_chunks * K

    @functools.partial(
        pl.kernel,
        out_type=jax.ShapeDtypeStruct((NC * N_ACC, W), jnp.float32),
        mesh=_MESH,
        scratch_types=[
            pltpu.VMEM_SHARED((N_ACC, W), jnp.float32),
            pltpu.VMEM((1, K), jnp.int32),
            pltpu.VMEM((1, K), jnp.int32),
            pltpu.VMEM((K, W), jnp.float32),
            pltpu.SemaphoreType.DMA,
        ],
    )
    def segsum(tab_hbm, idx_hbm, dst_hbm, zeros_hbm, out_hbm,
               acc, idxv, dstv, rows, sem):
        c = lax.axis_index("c")
        s = lax.axis_index("s")
        pltpu.sync_copy(zeros_hbm.at[pl.ds(s * RPT, RPT)],
                        acc.at[pl.ds(s * RPT, RPT)])
        plsc.subcore_barrier()
        tile_base = (c * NS + s) * per_tile

        def chunk(i, carry):
            base = tile_base + i * K
            pltpu.sync_copy(idx_hbm.at[pl.ds(base, K)], idxv.at[0])
            pltpu.sync_copy(dst_hbm.at[pl.ds(base, K)], dstv.at[0])
            pltpu.async_copy(tab_hbm.at[idxv.at[0]], rows, sem).wait()
            pltpu.sync_copy(rows, acc.at[dstv.at[0]], add=True)
            return carry

        lax.fori_loop(0, n_chunks, chunk, 0)
        plsc.subcore_barrier()
        pltpu.sync_copy(acc.at[pl.ds(s * RPT, RPT)],
                        out_hbm.at[pl.ds(c * N_ACC + s * RPT, RPT)])

    return segsum


_segsum_h = _make_segsum(D, CH_E)


def _layer_body(elu):
    def body(s0, s1, h, c0, c1, et, w1, b1, w2, b2, gm, bt, out):
        sl = et[4:5] + et[7:8] + et[14:15] + et[21:22]
        csum = c0[...] + c1[...]
        agg = (s0[...] + s1[...] + h[...] + sl
               + jnp.dot(csum, et[...], precision=lax.Precision.HIGHEST,
                         preferred_element_type=jnp.float32))
        hid = jnp.maximum(
            jnp.dot(agg, w1[...], preferred_element_type=jnp.float32)
            + b1[...], 0.0)
        hh = jnp.dot(hid, w2[...], preferred_element_type=jnp.float32) + b2[...]
        m = jnp.mean(hh, axis=0, keepdims=True)
        v = jnp.mean((hh - m) ** 2, axis=0, keepdims=True)
        y = (hh - m) * lax.rsqrt(v + 1e-5) * gm[...] + bt[...]
        if elu:
            y = jnp.where(y > 0, y, jnp.exp(y) - 1.0)
        out[...] = y
    return body


_layer_call = {
    flag: pl.pallas_call(
        _layer_body(flag),
        out_shape=jax.ShapeDtypeStruct((N, D), jnp.float32),
    )
    for flag in (True, False)
}


def kernel(x, edge_index, edge_attr, node_tabs, edge_tabs, W1, b1, W2, b2,
           gamma, beta):
    ntab = jnp.concatenate(
        [node_tabs.reshape(7 * 121, D), jnp.zeros((1, D), jnp.float32)], 0)
    nidx = x + (121 * jnp.arange(7, dtype=jnp.int32))[None, :]
    nidx = jnp.concatenate([nidx, jnp.full((N, 1), 847, jnp.int32)], 1)
    nidx = jnp.concatenate(
        [nidx, jnp.full((N_PAD - N, 8), 847, jnp.int32)], 0).reshape(-1)

    perm = jnp.argsort(edge_index[1])
    src = edge_index[0][perm]
    dst = edge_index[1][perm]
    edge_attr = edge_attr[perm]

    def _stride(a):
        return a.reshape(NW, K, CH_E).swapaxes(1, 2).reshape(-1)

    pw = jnp.array([1, 7, 49, 343], dtype=jnp.int32)
    p = jnp.arange(2401, dtype=jnp.int32)
    digs = (p[:, None] // pw[None, :]) % 7
    cols = 7 * jnp.arange(4, dtype=jnp.int32)[None, :] + digs
    ctab = jax.nn.one_hot(cols, D, dtype=jnp.float32).sum(1)
    ctab = jnp.concatenate([ctab, jnp.zeros((7, D), jnp.float32)], 0)
    cidx = (edge_attr * pw[None, :]).sum(1).astype(jnp.int32)
    cidx = _stride(jnp.concatenate(
        [cidx, jnp.full((E_PAD - E,), 2401, jnp.int32)]))

    srcp = _stride(jnp.concatenate([src, jnp.zeros((E_PAD - E,), jnp.int32)]))
    dstp = _stride(jnp.concatenate([dst, jnp.full((E_PAD - E,), N, jnp.int32)]))
    zerosD = jnp.zeros((N_ACC, D), jnp.float32)

    h = _embed(ntab, nidx)[:N]
    cc = _segsum_h(ctab, cidx, dstp, zerosD)
    c0 = cc[:N]
    c1 = cc[N_ACC:N_ACC + N]

    for l in range(L):
        ss = _segsum_h(h, srcp, dstp, zerosD)
        et = jnp.concatenate(
            [edge_tabs[l].reshape(28, D), jnp.zeros((100, D), jnp.float32)], 0)
        h = _layer_call[l < L - 1](
            ss[:N], ss[N_ACC:N_ACC + N], h, c0, c1, et,
            W1[l], b1[l].reshape(1, 2 * D), W2[l], b2[l].reshape(1, D),
            gamma[l].reshape(1, D), beta[l].reshape(1, D))
    return h


# speedup vs baseline: 7.0853x; 1.2753x over previous
import functools

import jax
import jax.numpy as jnp
from jax import lax
from jax.experimental import pallas as pl
from jax.experimental.pallas import tpu as pltpu
from jax.experimental.pallas import tpu_sc as plsc

N = 10000
D = 128
E = 320000
L = 5

NC = 2
NS = 16
NW = NC * NS

NPW = 320
N_PAD = NW * NPW

N_ACC = 10112
RPT = N_ACC // NS

K = 128

CH_E = 79
E_PAD = NW * CH_E * K

_MESH = plsc.VectorSubcoreMesh(core_axis_name="c", subcore_axis_name="s")


@functools.partial(
    pl.kernel,
    out_type=jax.ShapeDtypeStruct((N_PAD, D), jnp.float32),
    mesh=_MESH,
    scratch_types=[
        pltpu.VMEM((1, 128), jnp.int32),
        pltpu.VMEM((128, D), jnp.float32),
        pltpu.VMEM((16, D), jnp.float32),
        pltpu.SemaphoreType.DMA,
    ],
)
def _embed(tab_hbm, idx_hbm, out_hbm, idxv, rows, outv, sem):
    wid = lax.axis_index("s") * NC + lax.axis_index("c")

    def chunk(i, carry):
        node_off = wid * NPW + i * 16
        pltpu.sync_copy(idx_hbm.at[pl.ds(node_off * 8, 128)], idxv.at[0])
        pltpu.async_copy(tab_hbm.at[idxv.at[0]], rows, sem).wait()
        for j in range(16):
            for q in range(8):
                acc = rows[8 * j, pl.ds(16 * q, 16)]
                for t in range(1, 8):
                    acc = acc + rows[8 * j + t, pl.ds(16 * q, 16)]
                outv[j, pl.ds(16 * q, 16)] = acc
        pltpu.sync_copy(outv, out_hbm.at[pl.ds(node_off, 16)])
        return carry

    lax.fori_loop(0, NPW // 16, chunk, 0)


def _make_segsum(W, n_chunks):
    per_tile = n_chunks * K

    @functools.partial(
        pl.kernel,
        out_type=jax.ShapeDtypeStruct((NC * N_ACC, W), jnp.float32),
        mesh=_MESH,
        scratch_types=[
            pltpu.VMEM_SHARED((N_ACC, W), jnp.float32),
            pltpu.VMEM((2, K), jnp.int32),
            pltpu.VMEM((2, K), jnp.int32),
            pltpu.VMEM((2 * K, W), jnp.float32),
            pltpu.SemaphoreType.DMA,
            pltpu.SemaphoreType.DMA,
        ],
    )
    def segsum(tab_hbm, idx_hbm, dst_hbm, zeros_hbm, out_hbm,
               acc, idxv, dstv, rows, sem0, sem1):
        c = lax.axis_index("c")
        s = lax.axis_index("s")
        pltpu.sync_copy(zeros_hbm.at[pl.ds(s * RPT, RPT)],
                        acc.at[pl.ds(s * RPT, RPT)])
        plsc.subcore_barrier()
        tile_base = (c * NS + s) * per_tile
        sems = (sem0, sem1)

        def load(i, b):
            base = tile_base + i * K
            pltpu.sync_copy(idx_hbm.at[pl.ds(base, K)], idxv.at[b])
            pltpu.async_copy(tab_hbm.at[idxv.at[b]],
                             rows.at[pl.ds(b * K, K)], sems[b])
            pltpu.sync_copy(dst_hbm.at[pl.ds(base, K)], dstv.at[b])

        def drain_scatter(b):
            pltpu.make_async_copy(tab_hbm.at[idxv.at[b]],
                                  rows.at[pl.ds(b * K, K)], sems[b]).wait()
            pltpu.sync_copy(rows.at[pl.ds(b * K, K)],
                            acc.at[dstv.at[b]], add=True)

        load(0, 0)

        def pair(j, carry):
            i1 = 2 * j + 1

            @pl.when(i1 < n_chunks)
            def _():
                load(i1, 1)
            drain_scatter(0)

            @pl.when(i1 < n_chunks)
            def _():
                @pl.when(i1 + 1 < n_chunks)
                def _():
                    load(i1 + 1, 0)
                drain_scatter(1)
            return carry

        lax.fori_loop(0, (n_chunks + 1) // 2, pair, 0)
        plsc.subcore_barrier()
        pltpu.sync_copy(acc.at[pl.ds(s * RPT, RPT)],
                        out_hbm.at[pl.ds(c * N_ACC + s * RPT, RPT)])

    return segsum


_segsum_h = _make_segsum(D, CH_E)


def _layer_body(elu):
    def body(s0, s1, h, c0, c1, et, w1, b1, w2, b2, gm, bt, out):
        sl = et[4:5] + et[7:8] + et[14:15] + et[21:22]
        csum = c0[...] + c1[...]
        agg = (s0[...] + s1[...] + h[...] + sl
               + jnp.dot(csum, et[...], precision=lax.Precision.HIGHEST,
                         preferred_element_type=jnp.float32))
        hid = jnp.maximum(
            jnp.dot(agg, w1[...], preferred_element_type=jnp.float32)
            + b1[...], 0.0)
        hh = jnp.dot(hid, w2[...], preferred_element_type=jnp.float32) + b2[...]
        m = jnp.mean(hh, axis=0, keepdims=True)
        v = jnp.mean((hh - m) ** 2, axis=0, keepdims=True)
        y = (hh - m) * lax.rsqrt(v + 1e-5) * gm[...] + bt[...]
        if elu:
            y = jnp.where(y > 0, y, jnp.exp(y) - 1.0)
        out[...] = y
    return body


_layer_call = {
    flag: pl.pallas_call(
        _layer_body(flag),
        out_shape=jax.ShapeDtypeStruct((N, D), jnp.float32),
    )
    for flag in (True, False)
}


def kernel(x, edge_index, edge_attr, node_tabs, edge_tabs, W1, b1, W2, b2,
           gamma, beta):
    ntab = jnp.concatenate(
        [node_tabs.reshape(7 * 121, D), jnp.zeros((1, D), jnp.float32)], 0)
    nidx = x + (121 * jnp.arange(7, dtype=jnp.int32))[None, :]
    nidx = jnp.concatenate([nidx, jnp.full((N, 1), 847, jnp.int32)], 1)
    nidx = jnp.concatenate(
        [nidx, jnp.full((N_PAD - N, 8), 847, jnp.int32)], 0).reshape(-1)

    perm = jnp.argsort(edge_index[1])
    src = edge_index[0][perm]
    dst = edge_index[1][perm]
    edge_attr = edge_attr[perm]

    def _stride(a):
        return a.reshape(NW, K, CH_E).swapaxes(1, 2).reshape(-1)

    pw = jnp.array([1, 7, 49, 343], dtype=jnp.int32)
    p = jnp.arange(2401, dtype=jnp.int32)
    digs = (p[:, None] // pw[None, :]) % 7
    cols = 7 * jnp.arange(4, dtype=jnp.int32)[None, :] + digs
    ctab = jax.nn.one_hot(cols, D, dtype=jnp.float32).sum(1)
    ctab = jnp.concatenate([ctab, jnp.zeros((7, D), jnp.float32)], 0)
    cidx = (edge_attr * pw[None, :]).sum(1).astype(jnp.int32)
    cidx = _stride(jnp.concatenate(
        [cidx, jnp.full((E_PAD - E,), 2401, jnp.int32)]))

    srcp = _stride(jnp.concatenate([src, jnp.zeros((E_PAD - E,), jnp.int32)]))
    dstp = _stride(jnp.concatenate([dst, jnp.full((E_PAD - E,), N, jnp.int32)]))
    zerosD = jnp.zeros((N_ACC, D), jnp.float32)

    h = _embed(ntab, nidx)[:N]
    cc = _segsum_h(ctab, cidx, dstp, zerosD)
    c0 = cc[:N]
    c1 = cc[N_ACC:N_ACC + N]

    for l in range(L):
        ss = _segsum_h(h, srcp, dstp, zerosD)
        et = jnp.concatenate(
            [edge_tabs[l].reshape(28, D), jnp.zeros((100, D), jnp.float32)], 0)
        h = _layer_call[l < L - 1](
            ss[:N], ss[N_ACC:N_ACC + N], h, c0, c1, et,
            W1[l], b1[l].reshape(1, 2 * D), W2[l], b2[l].reshape(1, D),
            gamma[l].reshape(1, D), beta[l].reshape(1, D))
    return h
